# Initial kernel scaffold; baseline (speedup 1.0000x reference)
import jax
import jax.numpy as jnp
from jax.experimental import pallas as pl


def kernel(x, edge_index, flepe, Wq1, bq1, Wk1, bk1, Wv1, bv1, We1, Ws1, bs1, Wq2, bq2, Wk2, bk2, Wv2, bv2, We2, Ws2, bs2):
    raise NotImplementedError("write your pallas kernel here")


# speedup vs baseline: 3.2662x; 3.2662x over previous
import functools

import numpy as np
import jax
import jax.numpy as jnp
from jax import lax
from jax.experimental import pallas as pl
from jax.experimental.pallas import tpu as pltpu
from jax.experimental.pallas import tpu_sc as plsc

_N = 10000
_NP = 10240
_E = 320000
_D = 128
_DE = 16
_NC = 2
_NS = 16
_NW = _NC * _NS
_EPT = _E // _NW
_C = 80
_NCH = _EPT // _C
_SLICE = _NP // _NS
_INV_SQRT_D = 1.0 / np.sqrt(_D)

_mesh = plsc.VectorSubcoreMesh(core_axis_name="c", subcore_axis_name="s")


def _proj_body(x_ref, wq_ref, bq_ref, wk_ref, bk_ref, wv_ref, bv_ref,
               we_ref, ws_ref, bs_ref,
               q_ref, k_ref, v_ref, qe_ref, skip_ref):
    x = x_ref[...]
    q = (jnp.dot(x, wq_ref[...], preferred_element_type=jnp.float32)
         + bq_ref[...]) * _INV_SQRT_D
    q_ref[...] = q
    k_ref[...] = jnp.dot(x, wk_ref[...], preferred_element_type=jnp.float32) + bk_ref[...]
    v_ref[...] = jnp.dot(x, wv_ref[...], preferred_element_type=jnp.float32) + bv_ref[...]
    qe_ref[...] = lax.dot_general(q, we_ref[...], (((1,), (1,)), ((), ())),
                                  preferred_element_type=jnp.float32)
    skip_ref[...] = jnp.dot(x, ws_ref[...], preferred_element_type=jnp.float32) + bs_ref[...]


def _proj(x, Wq, bq, Wk, bk, Wv, bv, We, Ws, bs):
    n = x.shape[0]
    return pl.pallas_call(
        _proj_body,
        out_shape=[
            jax.ShapeDtypeStruct((n, _D), jnp.float32),
            jax.ShapeDtypeStruct((n, _D), jnp.float32),
            jax.ShapeDtypeStruct((n, _D), jnp.float32),
            jax.ShapeDtypeStruct((n, _DE), jnp.float32),
            jax.ShapeDtypeStruct((n, _D), jnp.float32),
        ],
    )(x, Wq, bq.reshape(1, _D), Wk, bk.reshape(1, _D), Wv, bv.reshape(1, _D),
      We, Ws, bs.reshape(1, _D))


def _epi_body(p0_ref, p1_ref, a0_ref, a1_ref, we_ref, skip_ref, out_ref,
              *, relu):
    a = a0_ref[...] + a1_ref[...]
    out = (p0_ref[...] + p1_ref[...]
           + jnp.dot(a, we_ref[...], preferred_element_type=jnp.float32)
           + skip_ref[...])
    if relu:
        out = jnp.maximum(out, 0.0)
    out_ref[...] = out


def _epi(p0, p1, a0, a1, We, skip, relu):
    n = skip.shape[0]
    return pl.pallas_call(
        functools.partial(_epi_body, relu=relu),
        out_shape=jax.ShapeDtypeStruct((n, _D), jnp.float32),
    )(p0, p1, a0, a1, We, skip)


def _sca_body(q_hbm, k_hbm, qe_hbm, fl_hbm, dst_hbm, src_hbm,
              wraw_hbm, s_hbm,
              dst_v, src_v, qv, kv, qev, fv, wv, s_loc, red_v, s_sh,
              sem_q, sem_k, sem_qe):
    cid = lax.axis_index("c")
    sid = lax.axis_index("s")
    wid = cid * _NS + sid
    base = wid * _EPT

    zero16 = jnp.zeros((16,), jnp.float32)

    def _zero(i, carry):
        s_loc[pl.ds(i * 16, 16)] = zero16
        return carry
    lax.fori_loop(0, _NP // 16, _zero, 0)

    lane = lax.iota(jnp.int32, 16)

    def _chunk(j, carry):
        eb = base + j * _C
        pltpu.sync_copy(dst_hbm.at[pl.ds(eb, _C)], dst_v)
        pltpu.sync_copy(src_hbm.at[pl.ds(eb, _C)], src_v)
        pltpu.sync_copy(fl_hbm.at[pl.ds(eb, _C)], fv)
        cq = pltpu.async_copy(q_hbm.at[dst_v], qv, sem_q)
        ck = pltpu.async_copy(k_hbm.at[src_v], kv, sem_k)
        ce = pltpu.async_copy(qe_hbm.at[dst_v], qev, sem_qe)
        cq.wait()
        ck.wait()
        ce.wait()

        def _grp(g, inner):
            rows = g * 16 + lane
            dots = jnp.zeros((16,), jnp.float32)
            for d in range(_D):
                cols = jnp.full((16,), d, jnp.int32)
                dots = dots + (plsc.load_gather(qv, [rows, cols])
                               * plsc.load_gather(kv, [rows, cols]))
            for d in range(_DE):
                cols = jnp.full((16,), d, jnp.int32)
                dots = dots + (plsc.load_gather(qev, [rows, cols])
                               * plsc.load_gather(fv, [rows, cols]))
            w16 = jnp.exp(dots)
            wv[pl.ds(g * 16, 16)] = w16
            d16 = dst_v[pl.ds(g * 16, 16)]
            for l in range(16):
                plsc.addupdate_scatter(s_loc, [d16], w16, mask=lane == l)
            return inner
        lax.fori_loop(0, _C // 16, _grp, 0)

        pltpu.sync_copy(wv, wraw_hbm.at[pl.ds(eb, _C)])
        return carry
    lax.fori_loop(0, _NCH, _chunk, 0)

    pltpu.sync_copy(s_loc, s_sh.at[sid])
    plsc.subcore_barrier()
    cslice = sid * _SLICE
    pltpu.sync_copy(s_sh.at[:, pl.ds(cslice, _SLICE)], red_v)

    def _red(g, carry):
        acc = red_v[0, pl.ds(g * 16, 16)]
        for r in range(1, _NS):
            acc = acc + red_v[r, pl.ds(g * 16, 16)]
        s_loc[pl.ds(g * 16, 16)] = acc
        return carry
    lax.fori_loop(0, _SLICE // 16, _red, 0)
    pltpu.sync_copy(s_loc.at[pl.ds(0, _SLICE)],
                    s_hbm.at[cid, pl.ds(cslice, _SLICE)])


def _sc_a(q, k, qe, flepe, dst, src):
    f = pl.kernel(
        _sca_body,
        out_type=[
            jax.ShapeDtypeStruct((_E,), jnp.float32),
            jax.ShapeDtypeStruct((_NC, _NP), jnp.float32),
        ],
        mesh=_mesh,
        compiler_params=pltpu.CompilerParams(needs_layout_passes=False, use_tc_tiling_on_sc=False),
        scratch_types=[
            pltpu.VMEM((_C,), jnp.int32),
            pltpu.VMEM((_C,), jnp.int32),
            pltpu.VMEM((_C, _D), jnp.float32),
            pltpu.VMEM((_C, _D), jnp.float32),
            pltpu.VMEM((_C, _DE), jnp.float32),
            pltpu.VMEM((_C, _DE), jnp.float32),
            pltpu.VMEM((_C,), jnp.float32),
            pltpu.VMEM((_NP,), jnp.float32),
            pltpu.VMEM((_NS, _SLICE), jnp.float32),
            pltpu.VMEM_SHARED((_NS, _NP), jnp.float32),
            pltpu.SemaphoreType.DMA,
            pltpu.SemaphoreType.DMA,
            pltpu.SemaphoreType.DMA,
        ],
    )
    return f(q, k, qe, flepe, dst, src)


def _scb_body(v_hbm, fl_hbm, dst_hbm, src_hbm, wraw_hbm, s_hbm,
              out_hbm, acc_hbm,
              dst_v, src_v, vv, fv, wv, sv, tv, out_sh, acc_sh, sem_v):
    cid = lax.axis_index("c")
    sid = lax.axis_index("s")
    wid = cid * _NS + sid
    base = wid * _EPT
    rbase = sid * _SLICE

    pltpu.sync_copy(s_hbm.at[0], sv)
    pltpu.sync_copy(s_hbm.at[1], tv)

    def _sum(g, carry):
        sl = pl.ds(g * 16, 16)
        sv[sl] = sv[sl] + tv[sl] + 1e-16
        return carry
    lax.fori_loop(0, _NP // 16, _sum, 0)

    zero16 = jnp.zeros((16,), jnp.float32)

    def _zv(i, carry):
        for b in range(_D // 16):
            vv[i, pl.ds(b * 16, 16)] = zero16
        fv[i, pl.ds(0, _DE)] = zero16
        return carry
    lax.fori_loop(0, _C, _zv, 0)
    for b in range(_SLICE // _C):
        pltpu.sync_copy(vv, out_sh.at[pl.ds(rbase + b * _C, _C)])
        pltpu.sync_copy(fv, acc_sh.at[pl.ds(rbase + b * _C, _C)])
    plsc.subcore_barrier()

    def _chunk(j, carry):
        eb = base + j * _C
        pltpu.sync_copy(dst_hbm.at[pl.ds(eb, _C)], dst_v)
        pltpu.sync_copy(src_hbm.at[pl.ds(eb, _C)], src_v)
        pltpu.sync_copy(fl_hbm.at[pl.ds(eb, _C)], fv)
        pltpu.sync_copy(wraw_hbm.at[pl.ds(eb, _C)], wv)
        cv = pltpu.async_copy(v_hbm.at[src_v], vv, sem_v)

        def _nrm(g, inner):
            sl = pl.ds(g * 16, 16)
            d16 = dst_v[sl]
            s16 = plsc.load_gather(sv, [d16])
            wv[sl] = wv[sl] / s16
            return inner
        lax.fori_loop(0, _C // 16, _nrm, 0)
        cv.wait()

        def _scale(g, inner):
            w16 = wv[pl.ds(g * 16, 16)]
            for l in range(16):
                i = g * 16 + l
                w = w16[l]
                for b in range(_D // 16):
                    sl = pl.ds(b * 16, 16)
                    vv[i, sl] = vv[i, sl] * w
                fv[i, pl.ds(0, _DE)] = fv[i, pl.ds(0, _DE)] * w
            return inner
        lax.fori_loop(0, _C // 16, _scale, 0)

        pltpu.sync_copy(vv, out_sh.at[dst_v], add=True)
        pltpu.sync_copy(fv, acc_sh.at[dst_v], add=True)
        return carry
    lax.fori_loop(0, _NCH, _chunk, 0)

    plsc.subcore_barrier()
    pltpu.sync_copy(out_sh.at[pl.ds(rbase, _SLICE)],
                    out_hbm.at[cid, pl.ds(rbase, _SLICE)])
    pltpu.sync_copy(acc_sh.at[pl.ds(rbase, _SLICE)],
                    acc_hbm.at[cid, pl.ds(rbase, _SLICE)])


def _sc_b(v, flepe, dst, src, wraw, s):
    f = pl.kernel(
        _scb_body,
        out_type=[
            jax.ShapeDtypeStruct((_NC, _NP, _D), jnp.float32),
            jax.ShapeDtypeStruct((_NC, _NP, _DE), jnp.float32),
        ],
        mesh=_mesh,
        compiler_params=pltpu.CompilerParams(needs_layout_passes=False, use_tc_tiling_on_sc=False),
        scratch_types=[
            pltpu.VMEM((_C,), jnp.int32),
            pltpu.VMEM((_C,), jnp.int32),
            pltpu.VMEM((_C, _D), jnp.float32),
            pltpu.VMEM((_C, _DE), jnp.float32),
            pltpu.VMEM((_C,), jnp.float32),
            pltpu.VMEM((_NP,), jnp.float32),
            pltpu.VMEM((_NP,), jnp.float32),
            pltpu.VMEM_SHARED((_NP, _D), jnp.float32),
            pltpu.VMEM_SHARED((_NP, _DE), jnp.float32),
            pltpu.SemaphoreType.DMA,
        ],
    )
    return f(v, flepe, dst, src, wraw, s)


def _layer(x, dst, src, flepe, Wq, bq, Wk, bk, Wv, bv, We, Ws, bs, relu):
    q, k, v, qe, skip = _proj(x, Wq, bq, Wk, bk, Wv, bv, We, Ws, bs)
    wraw, s = _sc_a(q, k, qe, flepe, dst, src)
    p, a = _sc_b(v, flepe, dst, src, wraw, s)
    return _epi(p[0, :_N], p[1, :_N], a[0, :_N], a[1, :_N], We, skip, relu)


def kernel(x, edge_index, flepe,
           Wq1, bq1, Wk1, bk1, Wv1, bv1, We1, Ws1, bs1,
           Wq2, bq2, Wk2, bk2, Wv2, bv2, We2, Ws2, bs2):
    src = edge_index[0]
    dst = edge_index[1]
    h = _layer(x, dst, src, flepe,
               Wq1, bq1, Wk1, bk1, Wv1, bv1, We1, Ws1, bs1, relu=True)
    return _layer(h, dst, src, flepe,
                  Wq2, bq2, Wk2, bk2, Wv2, bv2, We2, Ws2, bs2, relu=False)


# speedup vs baseline: 4.4684x; 1.3681x over previous
import functools

import numpy as np
import jax
import jax.numpy as jnp
from jax import lax
from jax.experimental import pallas as pl
from jax.experimental.pallas import tpu as pltpu
from jax.experimental.pallas import tpu_sc as plsc

_N = 10000
_NP = 10240
_E = 320000
_D = 128
_DE = 16
_NC = 2
_NS = 16
_NW = _NC * _NS
_EPT = _E // _NW
_C = 80
_NCH = _EPT // _C
_SLICE = _NP // _NS
_INV_SQRT_D = 1.0 / np.sqrt(_D)

_mesh = plsc.VectorSubcoreMesh(core_axis_name="c", subcore_axis_name="s")


def _proj_body(x_ref, wq_ref, bq_ref, wk_ref, bk_ref, wv_ref, bv_ref,
               we_ref, ws_ref, bs_ref,
               q_ref, k_ref, v_ref, qe_ref, skip_ref):
    x = x_ref[...]
    q = (jnp.dot(x, wq_ref[...], preferred_element_type=jnp.float32)
         + bq_ref[...]) * _INV_SQRT_D
    q_ref[...] = q
    k_ref[...] = jnp.dot(x, wk_ref[...], preferred_element_type=jnp.float32) + bk_ref[...]
    v_ref[...] = jnp.dot(x, wv_ref[...], preferred_element_type=jnp.float32) + bv_ref[...]
    qe_ref[...] = lax.dot_general(q, we_ref[...], (((1,), (1,)), ((), ())),
                                  preferred_element_type=jnp.float32)
    skip_ref[...] = jnp.dot(x, ws_ref[...], preferred_element_type=jnp.float32) + bs_ref[...]


def _proj(x, Wq, bq, Wk, bk, Wv, bv, We, Ws, bs):
    n = x.shape[0]
    return pl.pallas_call(
        _proj_body,
        out_shape=[
            jax.ShapeDtypeStruct((n, _D), jnp.float32),
            jax.ShapeDtypeStruct((n, _D), jnp.float32),
            jax.ShapeDtypeStruct((n, _D), jnp.float32),
            jax.ShapeDtypeStruct((n, _DE), jnp.float32),
            jax.ShapeDtypeStruct((n, _D), jnp.float32),
        ],
    )(x, Wq, bq.reshape(1, _D), Wk, bk.reshape(1, _D), Wv, bv.reshape(1, _D),
      We, Ws, bs.reshape(1, _D))


def _epi_body(p0_ref, p1_ref, a0_ref, a1_ref, we_ref, skip_ref, out_ref,
              *, relu):
    a = a0_ref[...] + a1_ref[...]
    out = (p0_ref[...] + p1_ref[...]
           + jnp.dot(a, we_ref[...], preferred_element_type=jnp.float32)
           + skip_ref[...])
    if relu:
        out = jnp.maximum(out, 0.0)
    out_ref[...] = out


def _epi(p0, p1, a0, a1, We, skip, relu):
    n = skip.shape[0]
    return pl.pallas_call(
        functools.partial(_epi_body, relu=relu),
        out_shape=jax.ShapeDtypeStruct((n, _D), jnp.float32),
    )(p0, p1, a0, a1, We, skip)


def _sca_body(q_hbm, k_hbm, qe_hbm, fl_hbm, dsts_hbm, srcs_hbm,
              wraw_hbm, s_hbm,
              dst_all, src_all, wv_all, s_loc, red_v,
              qv0, kv0, qev0, fv0, qv1, kv1, qev1, fv1,
              s_sh, sem0, sem1):
    cid = lax.axis_index("c")
    sid = lax.axis_index("s")
    wid = cid * _NS + sid
    base = wid * _EPT

    pltpu.sync_copy(dsts_hbm.at[wid], dst_all)
    pltpu.sync_copy(srcs_hbm.at[wid], src_all)

    zero16 = jnp.zeros((16,), jnp.float32)

    def _zero(i, carry):
        s_loc[pl.ds(i * 16, 16)] = zero16
        return carry
    lax.fori_loop(0, _NP // 16, _zero, 0)

    lane = lax.iota(jnp.int32, 16)
    bufs = ((qv0, kv0, qev0, fv0, sem0), (qv1, kv1, qev1, fv1, sem1))

    def _fire(j, b):
        qv, kv, qev, fv, sem = bufs[b]
        pltpu.async_copy(q_hbm.at[dst_all.at[j]], qv, sem)
        pltpu.async_copy(k_hbm.at[src_all.at[j]], kv, sem)
        pltpu.async_copy(qe_hbm.at[dst_all.at[j]], qev, sem)
        pltpu.async_copy(fl_hbm.at[pl.ds(base + j * _C, _C)], fv, sem)

    def _drain(j, b):
        qv, kv, qev, fv, sem = bufs[b]
        pltpu.make_async_copy(q_hbm.at[dst_all.at[j]], qv, sem).wait()
        pltpu.make_async_copy(k_hbm.at[src_all.at[j]], kv, sem).wait()
        pltpu.make_async_copy(qe_hbm.at[dst_all.at[j]], qev, sem).wait()
        pltpu.make_async_copy(fl_hbm.at[pl.ds(base + j * _C, _C)], fv, sem).wait()

    def _compute(j, b):
        qv, kv, qev, fv, sem = bufs[b]

        def _grp(g, inner):
            rows = g * 16 + lane
            dots = jnp.zeros((16,), jnp.float32)
            for d in range(_D):
                cols = jnp.full((16,), d, jnp.int32)
                dots = dots + (plsc.load_gather(qv, [rows, cols])
                               * plsc.load_gather(kv, [rows, cols]))
            for d in range(_DE):
                cols = jnp.full((16,), d, jnp.int32)
                dots = dots + (plsc.load_gather(qev, [rows, cols])
                               * plsc.load_gather(fv, [rows, cols]))
            w16 = jnp.exp(dots)
            wv_all[pl.ds(j * _C + g * 16, 16)] = w16
            d16 = dst_all[j, pl.ds(g * 16, 16)]
            for l in range(16):
                plsc.addupdate_scatter(s_loc, [d16], w16, mask=lane == l)
            return inner
        lax.fori_loop(0, _C // 16, _grp, 0)

    _fire(0, 0)

    def _pair(p, carry):
        for b2 in (0, 1):
            j = 2 * p + b2
            _fire(j + 1, 1 - b2)
            _drain(j, b2)
            _compute(j, b2)
        return carry
    lax.fori_loop(0, (_NCH - 1) // 2, _pair, 0)
    _drain(_NCH - 1, 0)
    _compute(_NCH - 1, 0)

    pltpu.sync_copy(wv_all, wraw_hbm.at[pl.ds(base, _EPT)])

    pltpu.sync_copy(s_loc, s_sh.at[sid])
    plsc.subcore_barrier()
    cslice = sid * _SLICE
    pltpu.sync_copy(s_sh.at[:, pl.ds(cslice, _SLICE)], red_v)

    def _red(g, carry):
        acc = red_v[0, pl.ds(g * 16, 16)]
        for r in range(1, _NS):
            acc = acc + red_v[r, pl.ds(g * 16, 16)]
        s_loc[pl.ds(g * 16, 16)] = acc
        return carry
    lax.fori_loop(0, _SLICE // 16, _red, 0)
    pltpu.sync_copy(s_loc.at[pl.ds(0, _SLICE)],
                    s_hbm.at[cid, pl.ds(cslice, _SLICE)])


def _sc_a(q, k, qe, flepe, dst3, src3):
    f = pl.kernel(
        _sca_body,
        out_type=[
            jax.ShapeDtypeStruct((_E,), jnp.float32),
            jax.ShapeDtypeStruct((_NC, _NP), jnp.float32),
        ],
        mesh=_mesh,
        compiler_params=pltpu.CompilerParams(needs_layout_passes=False, use_tc_tiling_on_sc=False),
        scratch_types=[
            pltpu.VMEM((_NCH, _C), jnp.int32),
            pltpu.VMEM((_NCH, _C), jnp.int32),
            pltpu.VMEM((_EPT,), jnp.float32),
            pltpu.VMEM((_NP,), jnp.float32),
            pltpu.VMEM((_NS, _SLICE), jnp.float32),
            pltpu.VMEM((_C, _D), jnp.float32),
            pltpu.VMEM((_C, _D), jnp.float32),
            pltpu.VMEM((_C, _DE), jnp.float32),
            pltpu.VMEM((_C, _DE), jnp.float32),
            pltpu.VMEM((_C, _D), jnp.float32),
            pltpu.VMEM((_C, _D), jnp.float32),
            pltpu.VMEM((_C, _DE), jnp.float32),
            pltpu.VMEM((_C, _DE), jnp.float32),
            pltpu.VMEM_SHARED((_NS, _NP), jnp.float32),
            pltpu.SemaphoreType.DMA,
            pltpu.SemaphoreType.DMA,
        ],
    )
    return f(q, k, qe, flepe, dst3, src3)


def _scb_body(v_hbm, fl_hbm, dst_hbm, src_hbm, wraw_hbm, s_hbm,
              out_hbm, acc_hbm,
              dst0, src0, wv0, vv0, fv0, dst1, src1, wv1, vv1, fv1,
              sv, tv,
              out_sh, acc_sh, semi0, semi1, semg0, semg1):
    cid = lax.axis_index("c")
    sid = lax.axis_index("s")
    wid = cid * _NS + sid
    base = wid * _EPT
    rbase = sid * _SLICE

    pltpu.sync_copy(s_hbm.at[0], sv)
    for c in range(_NS):
        pltpu.sync_copy(s_hbm.at[1, pl.ds(c * _SLICE, _SLICE)], tv)

        def _sum(g, carry, c=c):
            sl = pl.ds(c * _SLICE + g * 16, 16)
            sv[sl] = sv[sl] + tv[pl.ds(g * 16, 16)] + 1e-16
            return carry
        lax.fori_loop(0, _SLICE // 16, _sum, 0)

    zero16 = jnp.zeros((16,), jnp.float32)

    def _zv(i, carry):
        for b in range(_D // 16):
            vv0[i, pl.ds(b * 16, 16)] = zero16
        fv0[i, pl.ds(0, _DE)] = zero16
        return carry
    lax.fori_loop(0, _C, _zv, 0)
    for b in range(_SLICE // _C):
        pltpu.sync_copy(vv0, out_sh.at[pl.ds(rbase + b * _C, _C)])
        pltpu.sync_copy(fv0, acc_sh.at[pl.ds(rbase + b * _C, _C)])
    plsc.subcore_barrier()

    bufs = ((dst0, src0, wv0, vv0, fv0, semi0, semg0),
            (dst1, src1, wv1, vv1, fv1, semi1, semg1))

    def _fire_idx(j, b):
        dst_v, src_v, wv, vv, fv, semi, semg = bufs[b]
        pltpu.async_copy(dst_hbm.at[wid, j], dst_v, semi)
        pltpu.async_copy(src_hbm.at[wid, j], src_v, semi)
        pltpu.async_copy(wraw_hbm.at[pl.ds(base + j * _C, _C)], wv, semi)

    def _drain_idx(j, b):
        dst_v, src_v, wv, vv, fv, semi, semg = bufs[b]
        pltpu.make_async_copy(dst_hbm.at[wid, j], dst_v, semi).wait()
        pltpu.make_async_copy(src_hbm.at[wid, j], src_v, semi).wait()
        pltpu.make_async_copy(wraw_hbm.at[pl.ds(base + j * _C, _C)], wv, semi).wait()

    def _fire_gather(j, b):
        dst_v, src_v, wv, vv, fv, semi, semg = bufs[b]
        pltpu.async_copy(v_hbm.at[src_v], vv, semg)
        pltpu.async_copy(fl_hbm.at[pl.ds(base + j * _C, _C)], fv, semg)

    def _drain_gather(j, b):
        dst_v, src_v, wv, vv, fv, semi, semg = bufs[b]
        pltpu.make_async_copy(v_hbm.at[src_v], vv, semg).wait()
        pltpu.make_async_copy(fl_hbm.at[pl.ds(base + j * _C, _C)], fv, semg).wait()

    def _compute(j, b):
        dst_v, src_v, wv, vv, fv, semi, semg = bufs[b]

        def _scale(g, inner):
            sl = pl.ds(g * 16, 16)
            d16 = dst_v[sl]
            s16 = plsc.load_gather(sv, [d16])
            w16 = wv[sl] / s16
            for l in range(16):
                i = g * 16 + l
                w = w16[l]
                for bb in range(_D // 16):
                    sli = pl.ds(bb * 16, 16)
                    vv[i, sli] = vv[i, sli] * w
                fv[i, pl.ds(0, _DE)] = fv[i, pl.ds(0, _DE)] * w
            return inner
        lax.fori_loop(0, _C // 16, _scale, 0)

        pltpu.sync_copy(vv, out_sh.at[dst_v], add=True)
        pltpu.sync_copy(fv, acc_sh.at[dst_v], add=True)

    _fire_idx(0, 0)
    _drain_idx(0, 0)
    _fire_gather(0, 0)
    _fire_idx(1, 1)

    def _pair(p, carry):
        for b2 in (0, 1):
            j = 2 * p + b2
            jb = 1 - b2
            _drain_idx(j + 1, jb)
            _fire_gather(j + 1, jb)
            _drain_gather(j, b2)
            _compute(j, b2)

            @pl.when(j + 2 < _NCH)
            def _():
                _fire_idx(j + 2, b2)
        return carry
    lax.fori_loop(0, (_NCH - 1) // 2, _pair, 0)
    _drain_gather(_NCH - 1, 0)
    _compute(_NCH - 1, 0)

    plsc.subcore_barrier()
    pltpu.sync_copy(out_sh.at[pl.ds(rbase, _SLICE)],
                    out_hbm.at[cid, pl.ds(rbase, _SLICE)])
    pltpu.sync_copy(acc_sh.at[pl.ds(rbase, _SLICE)],
                    acc_hbm.at[cid, pl.ds(rbase, _SLICE)])


def _sc_b(v, flepe, dst, src, wraw, s):
    f = pl.kernel(
        _scb_body,
        out_type=[
            jax.ShapeDtypeStruct((_NC, _NP, _D), jnp.float32),
            jax.ShapeDtypeStruct((_NC, _NP, _DE), jnp.float32),
        ],
        mesh=_mesh,
        compiler_params=pltpu.CompilerParams(needs_layout_passes=False, use_tc_tiling_on_sc=False),
        scratch_types=[
            pltpu.VMEM((_C,), jnp.int32),
            pltpu.VMEM((_C,), jnp.int32),
            pltpu.VMEM((_C,), jnp.float32),
            pltpu.VMEM((_C, _D), jnp.float32),
            pltpu.VMEM((_C, _DE), jnp.float32),
            pltpu.VMEM((_C,), jnp.int32),
            pltpu.VMEM((_C,), jnp.int32),
            pltpu.VMEM((_C,), jnp.float32),
            pltpu.VMEM((_C, _D), jnp.float32),
            pltpu.VMEM((_C, _DE), jnp.float32),
            pltpu.VMEM((_NP,), jnp.float32),
            pltpu.VMEM((_SLICE,), jnp.float32),
            pltpu.VMEM_SHARED((_NP, _D), jnp.float32),
            pltpu.VMEM_SHARED((_NP, _DE), jnp.float32),
            pltpu.SemaphoreType.DMA,
            pltpu.SemaphoreType.DMA,
            pltpu.SemaphoreType.DMA,
            pltpu.SemaphoreType.DMA,
        ],
    )
    return f(v, flepe, dst, src, wraw, s)


def _layer(x, dst3, src3, flepe, Wq, bq, Wk, bk, Wv, bv, We, Ws, bs, relu):
    q, k, v, qe, skip = _proj(x, Wq, bq, Wk, bk, Wv, bv, We, Ws, bs)
    wraw, s = _sc_a(q, k, qe, flepe, dst3, src3)
    p, a = _sc_b(v, flepe, dst3, src3, wraw, s)
    return _epi(p[0, :_N], p[1, :_N], a[0, :_N], a[1, :_N], We, skip, relu)


def kernel(x, edge_index, flepe,
           Wq1, bq1, Wk1, bk1, Wv1, bv1, We1, Ws1, bs1,
           Wq2, bq2, Wk2, bk2, Wv2, bv2, We2, Ws2, bs2):
    src3 = edge_index[0].reshape(_NW, _NCH, _C)
    dst3 = edge_index[1].reshape(_NW, _NCH, _C)
    h = _layer(x, dst3, src3, flepe,
               Wq1, bq1, Wk1, bk1, Wv1, bv1, We1, Ws1, bs1, relu=True)
    return _layer(h, dst3, src3, flepe,
                  Wq2, bq2, Wk2, bk2, Wv2, bv2, We2, Ws2, bs2, relu=False)


# speedup vs baseline: 11.7611x; 2.6321x over previous
import functools

import numpy as np
import jax
import jax.numpy as jnp
from jax import lax
from jax.experimental import pallas as pl
from jax.experimental.pallas import tpu as pltpu
from jax.experimental.pallas import tpu_sc as plsc

_N = 10000
_NP = 10240
_E = 320000
_D = 128
_DE = 16
_NC = 2
_NS = 16
_NW = _NC * _NS
_EPT = _E // _NW
_C = 80
_NCH = _EPT // _C
_SLICE = _NP // _NS
_INV_SQRT_D = 1.0 / np.sqrt(_D)

_mesh = plsc.VectorSubcoreMesh(core_axis_name="c", subcore_axis_name="s")


def _proj_body(x_ref, wq_ref, bq_ref, wk_ref, bk_ref, wv_ref, bv_ref,
               we_ref, ws_ref, bs_ref,
               q_ref, k_ref, v_ref, qe_ref, skip_ref):
    x = x_ref[...]
    q = (jnp.dot(x, wq_ref[...], preferred_element_type=jnp.float32)
         + bq_ref[...]) * _INV_SQRT_D
    q_ref[...] = q
    k_ref[...] = jnp.dot(x, wk_ref[...], preferred_element_type=jnp.float32) + bk_ref[...]
    v_ref[...] = jnp.dot(x, wv_ref[...], preferred_element_type=jnp.float32) + bv_ref[...]
    qe_ref[...] = lax.dot_general(q, we_ref[...], (((1,), (1,)), ((), ())),
                                  preferred_element_type=jnp.float32)
    skip_ref[...] = jnp.dot(x, ws_ref[...], preferred_element_type=jnp.float32) + bs_ref[...]


def _proj(x, Wq, bq, Wk, bk, Wv, bv, We, Ws, bs):
    n = x.shape[0]
    return pl.pallas_call(
        _proj_body,
        out_shape=[
            jax.ShapeDtypeStruct((n, _D), jnp.float32),
            jax.ShapeDtypeStruct((n, _D), jnp.float32),
            jax.ShapeDtypeStruct((n, _D), jnp.float32),
            jax.ShapeDtypeStruct((n, _DE), jnp.float32),
            jax.ShapeDtypeStruct((n, _D), jnp.float32),
        ],
    )(x, Wq, bq.reshape(1, _D), Wk, bk.reshape(1, _D), Wv, bv.reshape(1, _D),
      We, Ws, bs.reshape(1, _D))


def _epi_body(p0_ref, p1_ref, a0_ref, a1_ref, we_ref, skip_ref, out_ref,
              *, relu):
    a = a0_ref[...] + a1_ref[...]
    out = (p0_ref[...] + p1_ref[...]
           + jnp.dot(a, we_ref[...], preferred_element_type=jnp.float32)
           + skip_ref[...])
    if relu:
        out = jnp.maximum(out, 0.0)
    out_ref[...] = out


def _epi(p0, p1, a0, a1, We, skip, relu):
    n = skip.shape[0]
    return pl.pallas_call(
        functools.partial(_epi_body, relu=relu),
        out_shape=jax.ShapeDtypeStruct((n, _D), jnp.float32),
    )(p0, p1, a0, a1, We, skip)


def _sca_body(q_hbm, k_hbm, qe_hbm, fl_hbm, dsts_hbm, srcs_hbm,
              wraw_hbm, s_hbm,
              dst_all, src_all, wv_all, s_loc, red_v,
              qv0, kv0, qev0, fv0, qv1, kv1, qev1, fv1,
              s_sh, sem0, sem1):
    cid = lax.axis_index("c")
    sid = lax.axis_index("s")
    wid = cid * _NS + sid
    base = wid * _EPT

    pltpu.sync_copy(dsts_hbm.at[wid], dst_all)
    pltpu.sync_copy(srcs_hbm.at[wid], src_all)

    zero16 = jnp.zeros((16,), jnp.float32)

    def _zero(i, carry):
        s_loc[pl.ds(i * 16, 16)] = zero16
        return carry
    lax.fori_loop(0, _NP // 16, _zero, 0)

    lane = lax.iota(jnp.int32, 16)
    bufs = ((qv0, kv0, qev0, fv0, sem0), (qv1, kv1, qev1, fv1, sem1))

    def _fire(j, b):
        qv, kv, qev, fv, sem = bufs[b]
        pltpu.async_copy(q_hbm.at[dst_all.at[j]], qv, sem)
        pltpu.async_copy(k_hbm.at[src_all.at[j]], kv, sem)
        pltpu.async_copy(qe_hbm.at[dst_all.at[j]], qev, sem)
        pltpu.async_copy(fl_hbm.at[pl.ds(base + j * _C, _C)], fv, sem)

    def _drain(j, b):
        qv, kv, qev, fv, sem = bufs[b]
        pltpu.make_async_copy(q_hbm.at[dst_all.at[j]], qv, sem).wait()
        pltpu.make_async_copy(k_hbm.at[src_all.at[j]], kv, sem).wait()
        pltpu.make_async_copy(qe_hbm.at[dst_all.at[j]], qev, sem).wait()
        pltpu.make_async_copy(fl_hbm.at[pl.ds(base + j * _C, _C)], fv, sem).wait()

    def _compute(j, b):
        qv, kv, qev, fv, sem = bufs[b]

        def _grp(g, inner):
            dots = jnp.zeros((16,), jnp.float32)
            for l in range(16):
                i = g * 16 + l
                acc = qev[i, pl.ds(0, _DE)] * fv[i, pl.ds(0, _DE)]
                for s in range(_D // 16):
                    sl = pl.ds(s * 16, 16)
                    acc = acc + qv[i, sl] * kv[i, sl]
                dot = jnp.sum(acc)
                dots = jnp.where(lane == l, dot, dots)
            w16 = jnp.exp(dots)
            wv_all[pl.ds(j * _C + g * 16, 16)] = w16
            d16 = dst_all[j, pl.ds(g * 16, 16)]
            for l in range(16):
                plsc.addupdate_scatter(s_loc, [d16], w16, mask=lane == l)
            return inner
        lax.fori_loop(0, _C // 16, _grp, 0)

    _fire(0, 0)

    def _pair(p, carry):
        for b2 in (0, 1):
            j = 2 * p + b2
            _fire(j + 1, 1 - b2)
            _drain(j, b2)
            _compute(j, b2)
        return carry
    lax.fori_loop(0, (_NCH - 1) // 2, _pair, 0)
    _drain(_NCH - 1, 0)
    _compute(_NCH - 1, 0)

    pltpu.sync_copy(wv_all, wraw_hbm.at[pl.ds(base, _EPT)])

    pltpu.sync_copy(s_loc, s_sh.at[sid])
    plsc.subcore_barrier()
    cslice = sid * _SLICE
    pltpu.sync_copy(s_sh.at[:, pl.ds(cslice, _SLICE)], red_v)

    def _red(g, carry):
        acc = red_v[0, pl.ds(g * 16, 16)]
        for r in range(1, _NS):
            acc = acc + red_v[r, pl.ds(g * 16, 16)]
        s_loc[pl.ds(g * 16, 16)] = acc
        return carry
    lax.fori_loop(0, _SLICE // 16, _red, 0)
    pltpu.sync_copy(s_loc.at[pl.ds(0, _SLICE)],
                    s_hbm.at[cid, pl.ds(cslice, _SLICE)])


def _sc_a(q, k, qe, flepe, dst3, src3):
    f = pl.kernel(
        _sca_body,
        out_type=[
            jax.ShapeDtypeStruct((_E,), jnp.float32),
            jax.ShapeDtypeStruct((_NC, _NP), jnp.float32),
        ],
        mesh=_mesh,
        compiler_params=pltpu.CompilerParams(needs_layout_passes=False, use_tc_tiling_on_sc=False),
        scratch_types=[
            pltpu.VMEM((_NCH, _C), jnp.int32),
            pltpu.VMEM((_NCH, _C), jnp.int32),
            pltpu.VMEM((_EPT,), jnp.float32),
            pltpu.VMEM((_NP,), jnp.float32),
            pltpu.VMEM((_NS, _SLICE), jnp.float32),
            pltpu.VMEM((_C, _D), jnp.float32),
            pltpu.VMEM((_C, _D), jnp.float32),
            pltpu.VMEM((_C, _DE), jnp.float32),
            pltpu.VMEM((_C, _DE), jnp.float32),
            pltpu.VMEM((_C, _D), jnp.float32),
            pltpu.VMEM((_C, _D), jnp.float32),
            pltpu.VMEM((_C, _DE), jnp.float32),
            pltpu.VMEM((_C, _DE), jnp.float32),
            pltpu.VMEM_SHARED((_NS, _NP), jnp.float32),
            pltpu.SemaphoreType.DMA,
            pltpu.SemaphoreType.DMA,
        ],
    )
    return f(q, k, qe, flepe, dst3, src3)


def _scb_body(v_hbm, fl_hbm, dst_hbm, src_hbm, wraw_hbm, s_hbm,
              out_hbm, acc_hbm,
              dst0, src0, wv0, vv0, fv0, dst1, src1, wv1, vv1, fv1,
              sv, tv,
              out_sh, acc_sh, semi0, semi1, semg0, semg1):
    cid = lax.axis_index("c")
    sid = lax.axis_index("s")
    wid = cid * _NS + sid
    base = wid * _EPT
    rbase = sid * _SLICE

    pltpu.sync_copy(s_hbm.at[0], sv)
    for c in range(_NS):
        pltpu.sync_copy(s_hbm.at[1, pl.ds(c * _SLICE, _SLICE)], tv)

        def _sum(g, carry, c=c):
            sl = pl.ds(c * _SLICE + g * 16, 16)
            sv[sl] = sv[sl] + tv[pl.ds(g * 16, 16)] + 1e-16
            return carry
        lax.fori_loop(0, _SLICE // 16, _sum, 0)

    zero16 = jnp.zeros((16,), jnp.float32)

    def _zv(i, carry):
        for b in range(_D // 16):
            vv0[i, pl.ds(b * 16, 16)] = zero16
        fv0[i, pl.ds(0, _DE)] = zero16
        return carry
    lax.fori_loop(0, _C, _zv, 0)
    for b in range(_SLICE // _C):
        pltpu.sync_copy(vv0, out_sh.at[pl.ds(rbase + b * _C, _C)])
        pltpu.sync_copy(fv0, acc_sh.at[pl.ds(rbase + b * _C, _C)])
    plsc.subcore_barrier()

    bufs = ((dst0, src0, wv0, vv0, fv0, semi0, semg0),
            (dst1, src1, wv1, vv1, fv1, semi1, semg1))

    def _fire_idx(j, b):
        dst_v, src_v, wv, vv, fv, semi, semg = bufs[b]
        pltpu.async_copy(dst_hbm.at[wid, j], dst_v, semi)
        pltpu.async_copy(src_hbm.at[wid, j], src_v, semi)
        pltpu.async_copy(wraw_hbm.at[pl.ds(base + j * _C, _C)], wv, semi)

    def _drain_idx(j, b):
        dst_v, src_v, wv, vv, fv, semi, semg = bufs[b]
        pltpu.make_async_copy(dst_hbm.at[wid, j], dst_v, semi).wait()
        pltpu.make_async_copy(src_hbm.at[wid, j], src_v, semi).wait()
        pltpu.make_async_copy(wraw_hbm.at[pl.ds(base + j * _C, _C)], wv, semi).wait()

    def _fire_gather(j, b):
        dst_v, src_v, wv, vv, fv, semi, semg = bufs[b]
        pltpu.async_copy(v_hbm.at[src_v], vv, semg)
        pltpu.async_copy(fl_hbm.at[pl.ds(base + j * _C, _C)], fv, semg)

    def _drain_gather(j, b):
        dst_v, src_v, wv, vv, fv, semi, semg = bufs[b]
        pltpu.make_async_copy(v_hbm.at[src_v], vv, semg).wait()
        pltpu.make_async_copy(fl_hbm.at[pl.ds(base + j * _C, _C)], fv, semg).wait()

    def _compute(j, b):
        dst_v, src_v, wv, vv, fv, semi, semg = bufs[b]

        def _scale(g, inner):
            sl = pl.ds(g * 16, 16)
            d16 = dst_v[sl]
            s16 = plsc.load_gather(sv, [d16])
            w16 = wv[sl] / s16
            for l in range(16):
                i = g * 16 + l
                w = w16[l]
                for bb in range(_D // 16):
                    sli = pl.ds(bb * 16, 16)
                    vv[i, sli] = vv[i, sli] * w
                fv[i, pl.ds(0, _DE)] = fv[i, pl.ds(0, _DE)] * w
            return inner
        lax.fori_loop(0, _C // 16, _scale, 0)

        pltpu.sync_copy(vv, out_sh.at[dst_v], add=True)
        pltpu.sync_copy(fv, acc_sh.at[dst_v], add=True)

    _fire_idx(0, 0)
    _drain_idx(0, 0)
    _fire_gather(0, 0)
    _fire_idx(1, 1)

    def _pair(p, carry):
        for b2 in (0, 1):
            j = 2 * p + b2
            jb = 1 - b2
            _drain_idx(j + 1, jb)
            _fire_gather(j + 1, jb)
            _drain_gather(j, b2)
            _compute(j, b2)

            @pl.when(j + 2 < _NCH)
            def _():
                _fire_idx(j + 2, b2)
        return carry
    lax.fori_loop(0, (_NCH - 1) // 2, _pair, 0)
    _drain_gather(_NCH - 1, 0)
    _compute(_NCH - 1, 0)

    plsc.subcore_barrier()
    pltpu.sync_copy(out_sh.at[pl.ds(rbase, _SLICE)],
                    out_hbm.at[cid, pl.ds(rbase, _SLICE)])
    pltpu.sync_copy(acc_sh.at[pl.ds(rbase, _SLICE)],
                    acc_hbm.at[cid, pl.ds(rbase, _SLICE)])


def _sc_b(v, flepe, dst, src, wraw, s):
    f = pl.kernel(
        _scb_body,
        out_type=[
            jax.ShapeDtypeStruct((_NC, _NP, _D), jnp.float32),
            jax.ShapeDtypeStruct((_NC, _NP, _DE), jnp.float32),
        ],
        mesh=_mesh,
        compiler_params=pltpu.CompilerParams(needs_layout_passes=False, use_tc_tiling_on_sc=False),
        scratch_types=[
            pltpu.VMEM((_C,), jnp.int32),
            pltpu.VMEM((_C,), jnp.int32),
            pltpu.VMEM((_C,), jnp.float32),
            pltpu.VMEM((_C, _D), jnp.float32),
            pltpu.VMEM((_C, _DE), jnp.float32),
            pltpu.VMEM((_C,), jnp.int32),
            pltpu.VMEM((_C,), jnp.int32),
            pltpu.VMEM((_C,), jnp.float32),
            pltpu.VMEM((_C, _D), jnp.float32),
            pltpu.VMEM((_C, _DE), jnp.float32),
            pltpu.VMEM((_NP,), jnp.float32),
            pltpu.VMEM((_SLICE,), jnp.float32),
            pltpu.VMEM_SHARED((_NP, _D), jnp.float32),
            pltpu.VMEM_SHARED((_NP, _DE), jnp.float32),
            pltpu.SemaphoreType.DMA,
            pltpu.SemaphoreType.DMA,
            pltpu.SemaphoreType.DMA,
            pltpu.SemaphoreType.DMA,
        ],
    )
    return f(v, flepe, dst, src, wraw, s)


def _layer(x, dst3, src3, flepe, Wq, bq, Wk, bk, Wv, bv, We, Ws, bs, relu):
    q, k, v, qe, skip = _proj(x, Wq, bq, Wk, bk, Wv, bv, We, Ws, bs)
    wraw, s = _sc_a(q, k, qe, flepe, dst3, src3)
    p, a = _sc_b(v, flepe, dst3, src3, wraw, s)
    return _epi(p[0, :_N], p[1, :_N], a[0, :_N], a[1, :_N], We, skip, relu)


def kernel(x, edge_index, flepe,
           Wq1, bq1, Wk1, bk1, Wv1, bv1, We1, Ws1, bs1,
           Wq2, bq2, Wk2, bk2, Wv2, bv2, We2, Ws2, bs2):
    src3 = edge_index[0].reshape(_NW, _NCH, _C)
    dst3 = edge_index[1].reshape(_NW, _NCH, _C)
    h = _layer(x, dst3, src3, flepe,
               Wq1, bq1, Wk1, bk1, Wv1, bv1, We1, Ws1, bs1, relu=True)
    return _layer(h, dst3, src3, flepe,
                  Wq2, bq2, Wk2, bk2, Wv2, bv2, We2, Ws2, bs2, relu=False)


# speedup vs baseline: 12.9532x; 1.1014x over previous
import functools

import numpy as np
import jax
import jax.numpy as jnp
from jax import lax
from jax.experimental import pallas as pl
from jax.experimental.pallas import tpu as pltpu
from jax.experimental.pallas import tpu_sc as plsc

_N = 10000
_NP = 10240
_E = 320000
_D = 128
_DE = 16
_NC = 2
_NS = 16
_NW = _NC * _NS
_EPT = _E // _NW
_C = 80
_NCH = _EPT // _C
_SLICE = _NP // _NS
_INV_SQRT_D = 1.0 / np.sqrt(_D)

_mesh = plsc.VectorSubcoreMesh(core_axis_name="c", subcore_axis_name="s")


def _proj_body(x_ref, wq_ref, bq_ref, wk_ref, bk_ref, wv_ref, bv_ref,
               we_ref, ws_ref, bs_ref,
               q_ref, k_ref, v_ref, qe_ref, skip_ref):
    x = x_ref[...]
    q = (jnp.dot(x, wq_ref[...], preferred_element_type=jnp.float32)
         + bq_ref[...]) * _INV_SQRT_D
    q_ref[...] = q
    k_ref[...] = jnp.dot(x, wk_ref[...], preferred_element_type=jnp.float32) + bk_ref[...]
    v_ref[...] = jnp.dot(x, wv_ref[...], preferred_element_type=jnp.float32) + bv_ref[...]
    qe_ref[...] = lax.dot_general(q, we_ref[...], (((1,), (1,)), ((), ())),
                                  preferred_element_type=jnp.float32)
    skip_ref[...] = jnp.dot(x, ws_ref[...], preferred_element_type=jnp.float32) + bs_ref[...]


def _proj(x, Wq, bq, Wk, bk, Wv, bv, We, Ws, bs):
    n = x.shape[0]
    return pl.pallas_call(
        _proj_body,
        out_shape=[
            jax.ShapeDtypeStruct((n, _D), jnp.float32),
            jax.ShapeDtypeStruct((n, _D), jnp.float32),
            jax.ShapeDtypeStruct((n, _D), jnp.float32),
            jax.ShapeDtypeStruct((n, _DE), jnp.float32),
            jax.ShapeDtypeStruct((n, _D), jnp.float32),
        ],
    )(x, Wq, bq.reshape(1, _D), Wk, bk.reshape(1, _D), Wv, bv.reshape(1, _D),
      We, Ws, bs.reshape(1, _D))


def _epi_body(p0_ref, p1_ref, a0_ref, a1_ref, we_ref, skip_ref, out_ref,
              *, relu):
    a = a0_ref[...] + a1_ref[...]
    out = (p0_ref[...] + p1_ref[...]
           + jnp.dot(a, we_ref[...], preferred_element_type=jnp.float32)
           + skip_ref[...])
    if relu:
        out = jnp.maximum(out, 0.0)
    out_ref[...] = out


def _epi(p0, p1, a0, a1, We, skip, relu):
    n = skip.shape[0]
    return pl.pallas_call(
        functools.partial(_epi_body, relu=relu),
        out_shape=jax.ShapeDtypeStruct((n, _D), jnp.float32),
    )(p0, p1, a0, a1, We, skip)


def _sca_body(q_hbm, k_hbm, qe_hbm, fl_hbm, dsts_hbm, srcs_hbm,
              wraw_hbm, s_hbm,
              dst_all, src_all, wv_all, s_loc, red_v,
              qv0, kv0, qev0, fv0, qv1, kv1, qev1, fv1,
              s_sh, sem0, sem1):
    cid = lax.axis_index("c")
    sid = lax.axis_index("s")
    wid = cid * _NS + sid
    base = wid * _EPT

    pltpu.sync_copy(dsts_hbm.at[wid], dst_all)
    pltpu.sync_copy(srcs_hbm.at[wid], src_all)

    zero16 = jnp.zeros((16,), jnp.float32)

    def _zero(i, carry):
        s_loc[pl.ds(i * 16, 16)] = zero16
        return carry
    lax.fori_loop(0, _NP // 16, _zero, 0)

    lane = lax.iota(jnp.int32, 16)
    bufs = ((qv0, kv0, qev0, fv0, sem0), (qv1, kv1, qev1, fv1, sem1))

    def _fire(j, b):
        qv, kv, qev, fv, sem = bufs[b]
        pltpu.async_copy(q_hbm.at[dst_all.at[j]], qv, sem)
        pltpu.async_copy(k_hbm.at[src_all.at[j]], kv, sem)
        pltpu.async_copy(qe_hbm.at[dst_all.at[j]], qev, sem)
        pltpu.async_copy(fl_hbm.at[pl.ds(base + j * _C, _C)], fv, sem)

    def _drain(j, b):
        qv, kv, qev, fv, sem = bufs[b]
        pltpu.make_async_copy(q_hbm.at[dst_all.at[j]], qv, sem).wait()
        pltpu.make_async_copy(k_hbm.at[src_all.at[j]], kv, sem).wait()
        pltpu.make_async_copy(qe_hbm.at[dst_all.at[j]], qev, sem).wait()
        pltpu.make_async_copy(fl_hbm.at[pl.ds(base + j * _C, _C)], fv, sem).wait()

    def _compute(j, b):
        qv, kv, qev, fv, sem = bufs[b]

        def _grp(g, inner):
            dots = jnp.zeros((16,), jnp.float32)
            for l in range(16):
                i = g * 16 + l
                acc = qev[i, pl.ds(0, _DE)] * fv[i, pl.ds(0, _DE)]
                for s in range(_D // 16):
                    sl = pl.ds(s * 16, 16)
                    acc = acc + qv[i, sl] * kv[i, sl]
                dot = jnp.sum(acc)
                dots = jnp.where(lane == l, dot, dots)
            w16 = jnp.exp(dots)
            wv_all[pl.ds(j * _C + g * 16, 16)] = w16
            d16 = dst_all[j, pl.ds(g * 16, 16)]
            for l in range(16):
                plsc.addupdate_scatter(s_loc, [d16], w16, mask=lane == l)
            return inner
        lax.fori_loop(0, _C // 16, _grp, 0)

    _fire(0, 0)

    def _pair(p, carry):
        for b2 in (0, 1):
            j = 2 * p + b2
            _fire(j + 1, 1 - b2)
            _drain(j, b2)
            _compute(j, b2)
        return carry
    lax.fori_loop(0, (_NCH - 1) // 2, _pair, 0)
    _drain(_NCH - 1, 0)
    _compute(_NCH - 1, 0)

    pltpu.sync_copy(wv_all, wraw_hbm.at[pl.ds(base, _EPT)])

    pltpu.sync_copy(s_loc, s_sh.at[sid])
    plsc.subcore_barrier()
    cslice = sid * _SLICE
    pltpu.sync_copy(s_sh.at[:, pl.ds(cslice, _SLICE)], red_v)

    def _red(g, carry):
        acc = red_v[0, pl.ds(g * 16, 16)]
        for r in range(1, _NS):
            acc = acc + red_v[r, pl.ds(g * 16, 16)]
        s_loc[pl.ds(g * 16, 16)] = acc
        return carry
    lax.fori_loop(0, _SLICE // 16, _red, 0)
    pltpu.sync_copy(s_loc.at[pl.ds(0, _SLICE)],
                    s_hbm.at[cid, pl.ds(cslice, _SLICE)])


def _sc_a(q, k, qe, flepe, dst3, src3):
    f = pl.kernel(
        _sca_body,
        out_type=[
            jax.ShapeDtypeStruct((_E,), jnp.float32),
            jax.ShapeDtypeStruct((_NC, _NP), jnp.float32),
        ],
        mesh=_mesh,
        compiler_params=pltpu.CompilerParams(needs_layout_passes=False, use_tc_tiling_on_sc=False),
        scratch_types=[
            pltpu.VMEM((_NCH, _C), jnp.int32),
            pltpu.VMEM((_NCH, _C), jnp.int32),
            pltpu.VMEM((_EPT,), jnp.float32),
            pltpu.VMEM((_NP,), jnp.float32),
            pltpu.VMEM((_NS, _SLICE), jnp.float32),
            pltpu.VMEM((_C, _D), jnp.float32),
            pltpu.VMEM((_C, _D), jnp.float32),
            pltpu.VMEM((_C, _DE), jnp.float32),
            pltpu.VMEM((_C, _DE), jnp.float32),
            pltpu.VMEM((_C, _D), jnp.float32),
            pltpu.VMEM((_C, _D), jnp.float32),
            pltpu.VMEM((_C, _DE), jnp.float32),
            pltpu.VMEM((_C, _DE), jnp.float32),
            pltpu.VMEM_SHARED((_NS, _NP), jnp.float32),
            pltpu.SemaphoreType.DMA,
            pltpu.SemaphoreType.DMA,
        ],
    )
    return f(q, k, qe, flepe, dst3, src3)


def _scb_body(v_hbm, fl_hbm, dst_hbm, src_hbm, wraw_hbm, s_hbm,
              out_hbm, acc_hbm,
              dst0, src0, wv0, vv0, fv0, ds0, dst1, src1, wv1, vv1, fv1, ds1,
              sv, tv,
              out_sh, acc_sh, semi0, semi1, semg0, semg1, sems0, sems1):
    cid = lax.axis_index("c")
    sid = lax.axis_index("s")
    wid = cid * _NS + sid
    base = wid * _EPT
    rbase = sid * _SLICE

    pltpu.sync_copy(s_hbm.at[0], sv)
    for c in range(_NS):
        pltpu.sync_copy(s_hbm.at[1, pl.ds(c * _SLICE, _SLICE)], tv)

        def _sum(g, carry, c=c):
            sl = pl.ds(c * _SLICE + g * 16, 16)
            sv[sl] = sv[sl] + tv[pl.ds(g * 16, 16)] + 1e-16
            return carry
        lax.fori_loop(0, _SLICE // 16, _sum, 0)

    zero16 = jnp.zeros((16,), jnp.float32)

    def _zv(i, carry):
        for b in range(_D // 16):
            vv0[i, pl.ds(b * 16, 16)] = zero16
        fv0[i, pl.ds(0, _DE)] = zero16
        return carry
    lax.fori_loop(0, _C, _zv, 0)
    for b in range(_SLICE // _C):
        pltpu.sync_copy(vv0, out_sh.at[pl.ds(rbase + b * _C, _C)])
        pltpu.sync_copy(fv0, acc_sh.at[pl.ds(rbase + b * _C, _C)])
    plsc.subcore_barrier()

    bufs = ((dst0, src0, wv0, vv0, fv0, ds0, semi0, semg0, sems0),
            (dst1, src1, wv1, vv1, fv1, ds1, semi1, semg1, sems1))

    def _fire_idx(j, b):
        dst_v, src_v, wv, vv, fv, dst_sc, semi, semg, sems = bufs[b]
        pltpu.async_copy(dst_hbm.at[wid, j], dst_v, semi)
        pltpu.async_copy(src_hbm.at[wid, j], src_v, semi)
        pltpu.async_copy(wraw_hbm.at[pl.ds(base + j * _C, _C)], wv, semi)

    def _drain_idx(j, b):
        dst_v, src_v, wv, vv, fv, dst_sc, semi, semg, sems = bufs[b]
        pltpu.make_async_copy(dst_hbm.at[wid, j], dst_v, semi).wait()
        pltpu.make_async_copy(src_hbm.at[wid, j], src_v, semi).wait()
        pltpu.make_async_copy(wraw_hbm.at[pl.ds(base + j * _C, _C)], wv, semi).wait()

    def _fire_gather(j, b):
        dst_v, src_v, wv, vv, fv, dst_sc, semi, semg, sems = bufs[b]
        pltpu.async_copy(v_hbm.at[src_v], vv, semg)
        pltpu.async_copy(fl_hbm.at[pl.ds(base + j * _C, _C)], fv, semg)

    def _drain_gather(j, b):
        dst_v, src_v, wv, vv, fv, dst_sc, semi, semg, sems = bufs[b]
        pltpu.make_async_copy(v_hbm.at[src_v], vv, semg).wait()
        pltpu.make_async_copy(fl_hbm.at[pl.ds(base + j * _C, _C)], fv, semg).wait()

    def _compute(j, b):
        dst_v, src_v, wv, vv, fv, dst_sc, semi, semg, sems = bufs[b]

        def _scale(g, inner):
            sl = pl.ds(g * 16, 16)
            d16 = dst_v[sl]
            dst_sc[sl] = d16
            s16 = plsc.load_gather(sv, [d16])
            w16 = wv[sl] / s16
            for l in range(16):
                i = g * 16 + l
                w = w16[l]
                for bb in range(_D // 16):
                    sli = pl.ds(bb * 16, 16)
                    vv[i, sli] = vv[i, sli] * w
                fv[i, pl.ds(0, _DE)] = fv[i, pl.ds(0, _DE)] * w
            return inner
        lax.fori_loop(0, _C // 16, _scale, 0)

    def _fire_scatter(j, b):
        dst_v, src_v, wv, vv, fv, dst_sc, semi, semg, sems = bufs[b]
        pltpu.async_copy(vv, out_sh.at[dst_sc], sems, add=True)
        pltpu.async_copy(fv, acc_sh.at[dst_sc], sems, add=True)

    def _drain_scatter(j, b):
        dst_v, src_v, wv, vv, fv, dst_sc, semi, semg, sems = bufs[b]
        pltpu.make_async_copy(vv, out_sh.at[dst_sc], sems).wait()
        pltpu.make_async_copy(fv, acc_sh.at[dst_sc], sems).wait()

    _fire_idx(0, 0)
    _drain_idx(0, 0)
    _fire_gather(0, 0)
    _fire_idx(1, 1)
    _drain_idx(1, 1)
    _fire_gather(1, 1)
    _drain_gather(0, 0)
    _compute(0, 0)
    _fire_scatter(0, 0)
    _fire_idx(2, 0)

    def _pair(p, carry):
        for b2 in (0, 1):
            j = 2 * p + 1 + b2
            pb = b2
            cb = 1 - b2
            _drain_scatter(j - 1, pb)
            _drain_idx(j + 1, pb)
            _fire_gather(j + 1, pb)
            _drain_gather(j, cb)
            _compute(j, cb)
            _fire_scatter(j, cb)
            _fire_idx(j + 2, cb)
        return carry
    lax.fori_loop(0, (_NCH - 3) // 2, _pair, 0)
    _drain_scatter(_NCH - 3, 0)
    _drain_idx(_NCH - 1, 0)
    _fire_gather(_NCH - 1, 0)
    _drain_gather(_NCH - 2, 1)
    _compute(_NCH - 2, 1)
    _fire_scatter(_NCH - 2, 1)
    _drain_scatter(_NCH - 2, 1)
    _drain_gather(_NCH - 1, 0)
    _compute(_NCH - 1, 0)
    _fire_scatter(_NCH - 1, 0)
    _drain_scatter(_NCH - 1, 0)

    plsc.subcore_barrier()
    pltpu.sync_copy(out_sh.at[pl.ds(rbase, _SLICE)],
                    out_hbm.at[cid, pl.ds(rbase, _SLICE)])
    pltpu.sync_copy(acc_sh.at[pl.ds(rbase, _SLICE)],
                    acc_hbm.at[cid, pl.ds(rbase, _SLICE)])


def _sc_b(v, flepe, dst, src, wraw, s):
    f = pl.kernel(
        _scb_body,
        out_type=[
            jax.ShapeDtypeStruct((_NC, _NP, _D), jnp.float32),
            jax.ShapeDtypeStruct((_NC, _NP, _DE), jnp.float32),
        ],
        mesh=_mesh,
        compiler_params=pltpu.CompilerParams(needs_layout_passes=False, use_tc_tiling_on_sc=False),
        scratch_types=[
            pltpu.VMEM((_C,), jnp.int32),
            pltpu.VMEM((_C,), jnp.int32),
            pltpu.VMEM((_C,), jnp.float32),
            pltpu.VMEM((_C, _D), jnp.float32),
            pltpu.VMEM((_C, _DE), jnp.float32),
            pltpu.VMEM((_C,), jnp.int32),
            pltpu.VMEM((_C,), jnp.int32),
            pltpu.VMEM((_C,), jnp.int32),
            pltpu.VMEM((_C,), jnp.float32),
            pltpu.VMEM((_C, _D), jnp.float32),
            pltpu.VMEM((_C, _DE), jnp.float32),
            pltpu.VMEM((_C,), jnp.int32),
            pltpu.VMEM((_NP,), jnp.float32),
            pltpu.VMEM((_SLICE,), jnp.float32),
            pltpu.VMEM_SHARED((_NP, _D), jnp.float32),
            pltpu.VMEM_SHARED((_NP, _DE), jnp.float32),
            pltpu.SemaphoreType.DMA,
            pltpu.SemaphoreType.DMA,
            pltpu.SemaphoreType.DMA,
            pltpu.SemaphoreType.DMA,
            pltpu.SemaphoreType.DMA,
            pltpu.SemaphoreType.DMA,
        ],
    )
    return f(v, flepe, dst, src, wraw, s)


def _layer(x, dst3, src3, flepe, Wq, bq, Wk, bk, Wv, bv, We, Ws, bs, relu):
    q, k, v, qe, skip = _proj(x, Wq, bq, Wk, bk, Wv, bv, We, Ws, bs)
    wraw, s = _sc_a(q, k, qe, flepe, dst3, src3)
    p, a = _sc_b(v, flepe, dst3, src3, wraw, s)
    return _epi(p[0, :_N], p[1, :_N], a[0, :_N], a[1, :_N], We, skip, relu)


def kernel(x, edge_index, flepe,
           Wq1, bq1, Wk1, bk1, Wv1, bv1, We1, Ws1, bs1,
           Wq2, bq2, Wk2, bk2, Wv2, bv2, We2, Ws2, bs2):
    src3 = edge_index[0].reshape(_NW, _NCH, _C)
    dst3 = edge_index[1].reshape(_NW, _NCH, _C)
    h = _layer(x, dst3, src3, flepe,
               Wq1, bq1, Wk1, bk1, Wv1, bv1, We1, Ws1, bs1, relu=True)
    return _layer(h, dst3, src3, flepe,
                  Wq2, bq2, Wk2, bk2, Wv2, bv2, We2, Ws2, bs2, relu=False)
